# Initial kernel scaffold; baseline (speedup 1.0000x reference)
import functools

import jax
import jax.numpy as jnp
from jax import lax
from jax.experimental import pallas as pl
from jax.experimental.pallas import tpu as pltpu
from jax.experimental.pallas import tpu_sc as plsc

NUM_TOK = 16384 * 50
DIM = 64
NC = 2
NS = 16
NW = NC * NS
PER_W = NUM_TOK // NW
CHUNK = 128
NCHUNK = PER_W // CHUNK

_mesh = plsc.VectorSubcoreMesh(core_axis_name="c", subcore_axis_name="s")


@functools.partial(
    pl.kernel,
    mesh=_mesh,
    out_type=jax.ShapeDtypeStruct((NUM_TOK, DIM), jnp.float32),
    scratch_types=[
        pltpu.VMEM((CHUNK,), jnp.int32),
        pltpu.VMEM((CHUNK, DIM), jnp.float32),
        pltpu.SemaphoreType.DMA,
    ],
)
def _gather(idx_hbm, table_hbm, out_hbm, idx_v, rows_v, sem):
    wid = lax.axis_index("s") * NC + lax.axis_index("c")
    base = wid * PER_W

    def body(i, carry):
        off = base + i * CHUNK
        pltpu.sync_copy(idx_hbm.at[pl.ds(off, CHUNK)], idx_v)
        pltpu.async_copy(table_hbm.at[idx_v], rows_v, sem).wait()
        pltpu.sync_copy(rows_v, out_hbm.at[pl.ds(off, CHUNK)])
        return carry

    lax.fori_loop(0, NCHUNK, body, 0)


def kernel(tokens_ids, embedding_tensor):
    flat = tokens_ids.reshape(-1).astype(jnp.int32)
    out = _gather(flat, embedding_tensor)
    return out.reshape(*tokens_ids.shape, DIM)


# speedup vs baseline: 1.5748x; 1.5748x over previous
import functools

import jax
import jax.numpy as jnp
from jax import lax
from jax.experimental import pallas as pl
from jax.experimental.pallas import tpu as pltpu
from jax.experimental.pallas import tpu_sc as plsc

NUM_TOK = 16384 * 50
DIM = 64
NC = 2
NS = 16
NW = NC * NS
PER_W = NUM_TOK // NW
CHUNK = 128
NCHUNK = PER_W // CHUNK

_mesh = plsc.VectorSubcoreMesh(core_axis_name="c", subcore_axis_name="s")


@functools.partial(
    pl.kernel,
    mesh=_mesh,
    out_type=jax.ShapeDtypeStruct((NUM_TOK, DIM), jnp.float32),
    scratch_types=[
        pltpu.VMEM((CHUNK,), jnp.int32),
        pltpu.VMEM((CHUNK, DIM), jnp.float32),
        pltpu.SemaphoreType.DMA,
    ],
    compiler_params=pltpu.CompilerParams(use_tc_tiling_on_sc=False),
)
def _gather(idx_hbm, table_hbm, out_hbm, idx_v, rows_v, sem):
    wid = lax.axis_index("s") * NC + lax.axis_index("c")
    base = wid * PER_W

    def body(i, carry):
        off = base + i * CHUNK
        pltpu.sync_copy(idx_hbm.at[pl.ds(off, CHUNK)], idx_v)
        pltpu.async_copy(table_hbm.at[idx_v], rows_v, sem).wait()
        pltpu.sync_copy(rows_v, out_hbm.at[pl.ds(off, CHUNK)])
        return carry

    lax.fori_loop(0, NCHUNK, body, 0)


def kernel(tokens_ids, embedding_tensor):
    flat = tokens_ids.reshape(-1).astype(jnp.int32)
    out = _gather(flat, embedding_tensor)
    return out.reshape(*tokens_ids.shape, DIM)


# speedup vs baseline: 1.8621x; 1.1824x over previous
import functools

import jax
import jax.numpy as jnp
from jax import lax
from jax.experimental import pallas as pl
from jax.experimental.pallas import tpu as pltpu
from jax.experimental.pallas import tpu_sc as plsc

NUM_TOK = 16384 * 50
DIM = 64
NC = 2
NS = 16
NW = NC * NS
PER_W = NUM_TOK // NW
SUB = 128
C = 512
NSUB = C // SUB
NCH = PER_W // C
ROW_BYTES = C * DIM * 4

_mesh = plsc.VectorSubcoreMesh(core_axis_name="c", subcore_axis_name="s")


@functools.partial(
    pl.kernel,
    mesh=_mesh,
    out_type=jax.ShapeDtypeStruct((NUM_TOK, DIM), jnp.float32),
    scratch_types=[
        pltpu.VMEM((PER_W,), jnp.int32),
        pltpu.VMEM((C, DIM), jnp.float32),
        pltpu.VMEM((C, DIM), jnp.float32),
        pltpu.SemaphoreType.DMA,
        pltpu.SemaphoreType.DMA,
        pltpu.SemaphoreType.DMA,
        pltpu.SemaphoreType.DMA,
    ],
    compiler_params=pltpu.CompilerParams(use_tc_tiling_on_sc=False),
)
def _gather(idx_hbm, table_hbm, out_hbm, idx_v, rows0, rows1,
            gsem0, gsem1, ssem0, ssem1):
    wid = lax.axis_index("s") * NC + lax.axis_index("c")
    base = wid * PER_W
    rows = (rows0, rows1)
    gsem = (gsem0, gsem1)
    ssem = (ssem0, ssem1)

    pltpu.sync_copy(idx_hbm.at[pl.ds(base, PER_W)], idx_v)

    def fire_gather(c, b):
        for j in range(NSUB):
            pltpu.async_copy(
                table_hbm.at[idx_v.at[pl.ds(c * C + j * SUB, SUB)]],
                rows[b].at[pl.ds(j * SUB, SUB)],
                gsem[b],
            )

    def wait_sem(sem, b):
        pltpu.make_async_copy(
            table_hbm.at[pl.ds(0, C)], rows[b], sem[b]
        ).wait()

    def fire_store(c, b):
        pltpu.async_copy(rows[b], out_hbm.at[pl.ds(base + c * C, C)], ssem[b])

    def body(g, carry):
        for b in (0, 1):
            c = 2 * g + b
            @pl.when(g >= 1)
            def _():
                wait_sem(ssem, b)
            fire_gather(c, b)
        for b in (0, 1):
            c = 2 * g + b
            wait_sem(gsem, b)
            fire_store(c, b)
        return carry

    lax.fori_loop(0, NCH // 2, body, 0)
    wait_sem(ssem, 0)
    wait_sem(ssem, 1)


def kernel(tokens_ids, embedding_tensor):
    flat = tokens_ids.reshape(-1).astype(jnp.int32)
    out = _gather(flat, embedding_tensor)
    return out.reshape(*tokens_ids.shape, DIM)


# speedup vs baseline: 1.8649x; 1.0015x over previous
import functools

import jax
import jax.numpy as jnp
from jax import lax
from jax.experimental import pallas as pl
from jax.experimental.pallas import tpu as pltpu
from jax.experimental.pallas import tpu_sc as plsc

NUM_TOK = 16384 * 50
DIM = 64
NC = 2
NS = 16
NW = NC * NS
PER_W = NUM_TOK // NW
SUB = 512
C = 512
NSUB = C // SUB
NCH = PER_W // C
ROW_BYTES = C * DIM * 4

_mesh = plsc.VectorSubcoreMesh(core_axis_name="c", subcore_axis_name="s")


@functools.partial(
    pl.kernel,
    mesh=_mesh,
    out_type=jax.ShapeDtypeStruct((NUM_TOK, DIM), jnp.float32),
    scratch_types=[
        pltpu.VMEM((PER_W,), jnp.int32),
        pltpu.VMEM((C, DIM), jnp.float32),
        pltpu.VMEM((C, DIM), jnp.float32),
        pltpu.SemaphoreType.DMA,
        pltpu.SemaphoreType.DMA,
        pltpu.SemaphoreType.DMA,
        pltpu.SemaphoreType.DMA,
    ],
    compiler_params=pltpu.CompilerParams(use_tc_tiling_on_sc=False),
)
def _gather(idx_hbm, table_hbm, out_hbm, idx_v, rows0, rows1,
            gsem0, gsem1, ssem0, ssem1):
    wid = lax.axis_index("s") * NC + lax.axis_index("c")
    base = wid * PER_W
    rows = (rows0, rows1)
    gsem = (gsem0, gsem1)
    ssem = (ssem0, ssem1)

    pltpu.sync_copy(idx_hbm.at[pl.ds(base, PER_W)], idx_v)

    def fire_gather(c, b):
        for j in range(NSUB):
            pltpu.async_copy(
                table_hbm.at[idx_v.at[pl.ds(c * C + j * SUB, SUB)]],
                rows[b].at[pl.ds(j * SUB, SUB)],
                gsem[b],
            )

    def wait_sem(sem, b):
        pltpu.make_async_copy(
            table_hbm.at[pl.ds(0, C)], rows[b], sem[b]
        ).wait()

    def fire_store(c, b):
        pltpu.async_copy(rows[b], out_hbm.at[pl.ds(base + c * C, C)], ssem[b])

    def body(g, carry):
        for b in (0, 1):
            c = 2 * g + b
            @pl.when(g >= 1)
            def _():
                wait_sem(ssem, b)
            fire_gather(c, b)
        for b in (0, 1):
            c = 2 * g + b
            wait_sem(gsem, b)
            fire_store(c, b)
        return carry

    lax.fori_loop(0, NCH // 2, body, 0)
    wait_sem(ssem, 0)
    wait_sem(ssem, 1)


def kernel(tokens_ids, embedding_tensor):
    flat = tokens_ids.reshape(-1).astype(jnp.int32)
    out = _gather(flat, embedding_tensor)
    return out.reshape(*tokens_ids.shape, DIM)
